# baseline (speedup 1.0000x reference)
import functools

import jax
import jax.numpy as jnp
from jax import lax
from jax.experimental import pallas as pl
from jax.experimental.pallas import tpu as pltpu
from jax.experimental.pallas import tpu_sc as plsc

L = 16
NC, NS = 2, 16
NW = NC * NS
C = 2048
ROWS = 16
ROW = C // ROWS


def _sc_lookup(xyz_flat, words, par, n_points, g0, g1, g2):
    pw = n_points // NW
    nch = pw // C
    hi1 = g1 * g2
    hi2 = g2
    n_words = (g0 * g1 * g2) // 4
    fhi0 = float(g0) - 0.5
    fhi1 = float(g1) - 0.5
    fhi2 = float(g2) - 0.5

    mesh = plsc.VectorSubcoreMesh(core_axis_name="c", subcore_axis_name="s")

    @functools.partial(
        pl.kernel,
        mesh=mesh,
        out_type=jax.ShapeDtypeStruct((n_points // 4,), jnp.int32),
        compiler_params=pltpu.CompilerParams(needs_layout_passes=False),
        scratch_types=[
            pltpu.VMEM((3 * C,), jnp.float32),
            pltpu.VMEM((ROWS, ROW), jnp.int32),
            pltpu.VMEM((C,), jnp.int32),
            pltpu.VMEM((C,), jnp.int32),
            pltpu.VMEM((C // 4,), jnp.int32),
            pltpu.VMEM((L,), jnp.float32),
            pltpu.SemaphoreType.DMA,
        ],
    )
    def k(xyz_hbm, words_hbm, par_hbm, out_hbm,
          xyz_v, idx_v, gath_v, aux_v, out_v, par_v, sem):
        wid = lax.axis_index("s") * NC + lax.axis_index("c")
        pltpu.sync_copy(par_hbm, par_v)
        pars = par_v[...]
        sx, sy, sz = pars[0], pars[1], pars[2]
        hx, hy, hz = pars[3], pars[4], pars[5]
        base0 = wid * pw
        lanes = lax.iota(jnp.int32, L)

        def chunk_body(t, _):
            base = base0 + t * C
            pltpu.sync_copy(
                xyz_hbm.at[pl.ds(pl.multiple_of(base * 3, 8), 3 * C)], xyz_v)

            def row_a(j, _):
                for o in range(ROW // L):
                    p3 = (j * ROW + o * L) * 3 + lanes * 3
                    x = plsc.load_gather(xyz_v, [p3])
                    y = plsc.load_gather(xyz_v, [p3 + 1])
                    z = plsc.load_gather(xyz_v, [p3 + 2])
                    vx = x * sx + hx
                    vy = y * sy + hy
                    vz = z * sz + hz
                    inb = ((vx >= -0.5) & (vx < fhi0)
                           & (vy >= -0.5) & (vy < fhi1)
                           & (vz >= -0.5) & (vz < fhi2))
                    ix = (vx + 0.5).astype(jnp.int32)
                    iy = (vy + 0.5).astype(jnp.int32)
                    iz = (vz + 0.5).astype(jnp.int32)
                    lin = ix * hi1 + iy * hi2 + iz
                    widx = jnp.minimum(jnp.maximum(lin >> 2, 0), n_words - 1)
                    aux = ((lin & 3) << 3) | jnp.where(inb, 32, 0)
                    idx_v[j, pl.ds(o * L, L)] = widx
                    aux_v[pl.ds(j * ROW + o * L, L)] = aux
                return 0

            lax.fori_loop(0, ROWS, row_a, 0)

            copies = [
                pltpu.async_copy(words_hbm.at[idx_v.at[j]],
                                 gath_v.at[pl.ds(j * ROW, ROW)], sem)
                for j in range(ROWS)
            ]
            for cp in copies:
                cp.wait()

            def row_c(j, _):
                for h in range(ROW // (4 * L)):
                    pb = j * ROW + h * 4 * L
                    bits = []
                    for s in range(4):
                        pidx = pb + s + lanes * 4
                        w = plsc.load_gather(gath_v, [pidx])
                        a = plsc.load_gather(aux_v, [pidx])
                        bits.append((w >> (a & 31)) & (a >> 5))
                    packed = (bits[0] | (bits[1] << 8)
                              | (bits[2] << 16) | (bits[3] << 24))
                    out_v[pl.ds((pb >> 2), L)] = packed
                return 0

            lax.fori_loop(0, ROWS, row_c, 0)
            pltpu.sync_copy(
                out_v,
                out_hbm.at[pl.ds(pl.multiple_of(base0 // 4 + t * (C // 4), 8),
                                 C // 4)])
            return 0

        lax.fori_loop(0, nch, chunk_body, 0)

    return k(xyz_flat, words, par)


def kernel(xyz, mask, xyz2ijk_scale, xyz2ijk_shift):
    g0, g1, g2 = mask.shape
    shape = xyz.shape[:-1]
    n_points = xyz.size // 3
    xyz_flat = xyz.reshape(-1)
    words = jax.lax.bitcast_convert_type(
        mask.astype(jnp.uint8).reshape(-1, 4), jnp.int32)
    par = jnp.concatenate(
        [xyz2ijk_scale.astype(jnp.float32),
         xyz2ijk_shift.astype(jnp.float32),
         jnp.zeros((10,), jnp.float32)])
    out_words = _sc_lookup(xyz_flat, words, par, n_points, g0, g1, g2)
    out_bytes = jax.lax.bitcast_convert_type(out_words, jnp.uint8)
    return out_bytes.reshape(shape).astype(jnp.bool_)
